# Initial kernel scaffold; baseline (speedup 1.0000x reference)
import jax
import jax.numpy as jnp
from jax.experimental import pallas as pl


def kernel(x, edge_index, batch, W_enc, b_enc, W_convs, b_convs, W_proj, b_proj, ln_g, ln_b):
    raise NotImplementedError("write your pallas kernel here")


# speedup vs baseline: 10.8426x; 10.8426x over previous
import functools

import jax
import jax.numpy as jnp
from jax import lax
from jax.experimental import pallas as pl
from jax.experimental.pallas import tpu as pltpu
from jax.experimental.pallas import tpu_sc as plsc

_N = 10000
_H = 128
_E = 320000
_G = 64
_D = 64

_NC = 2
_NS = 16
_NW = _NC * _NS
_EPW = _E // _NW
_C = 80
_NCHUNK = _EPW // _C
_RA = 632
_RL = 520
_ZR = 8

_BN = 1000
_NB = _N // _BN

_mesh = plsc.VectorSubcoreMesh(core_axis_name="c", subcore_axis_name="s")


@functools.partial(
    pl.kernel,
    mesh=_mesh,
    out_type=jax.ShapeDtypeStruct((_NC, _N, _H), jnp.float32),
    scratch_types=[
        pltpu.VMEM((_C,), jnp.int32),
        pltpu.VMEM((_C, _H), jnp.float32),
        pltpu.VMEM((_ZR, _H), jnp.float32),
        pltpu.VMEM_SHARED((_N, _H), jnp.float32),
    ],
)
def _deg_kernel(dst_hbm, out_hbm, idx_v, ones_v, zbuf_v, acc_sh):
    c = lax.axis_index("c")
    s = lax.axis_index("s")
    wid = c * _NS + s
    base = s * _RA

    one16 = jnp.ones((16,), jnp.float32)
    zero16 = jnp.zeros((16,), jnp.float32)

    def _fill_ones(i, carry):
        for j in range(_H // 16):
            ones_v[i, pl.ds(j * 16, 16)] = one16
        return carry

    lax.fori_loop(0, _C, _fill_ones, 0)

    def _fill_zero(i, carry):
        for j in range(_H // 16):
            zbuf_v[i, pl.ds(j * 16, 16)] = zero16
        return carry

    lax.fori_loop(0, _ZR, _fill_zero, 0)

    def _zero_copy(i, carry):
        pltpu.sync_copy(zbuf_v, acc_sh.at[pl.ds(base + i * _ZR, _ZR)])
        return carry

    lax.fori_loop(0, _RL // _ZR, _zero_copy, 0)

    @pl.when(s < _NS - 1)
    def _zero_tail():
        def _zc(i, carry):
            pltpu.sync_copy(zbuf_v,
                            acc_sh.at[pl.ds(base + _RL + i * _ZR, _ZR)])
            return carry
        lax.fori_loop(0, (_RA - _RL) // _ZR, _zc, 0)

    plsc.subcore_barrier()

    def _chunk(k, carry):
        off = wid * _EPW + k * _C
        pltpu.sync_copy(dst_hbm.at[pl.ds(off, _C)], idx_v)
        pltpu.sync_copy(ones_v, acc_sh.at[idx_v], add=True)
        return carry

    lax.fori_loop(0, _NCHUNK, _chunk, 0)

    plsc.subcore_barrier()
    pltpu.sync_copy(acc_sh.at[pl.ds(base, _RL)],
                    out_hbm.at[c, pl.ds(base, _RL)])

    @pl.when(s < _NS - 1)
    def _out_tail():
        pltpu.sync_copy(acc_sh.at[pl.ds(base + _RL, _RA - _RL)],
                        out_hbm.at[c, pl.ds(base + _RL, _RA - _RL)])


@functools.partial(
    pl.kernel,
    mesh=_mesh,
    out_type=jax.ShapeDtypeStruct((_NC, _N, _H), jnp.float32),
    scratch_types=[
        pltpu.VMEM((_C,), jnp.int32),
        pltpu.VMEM((_C,), jnp.int32),
        pltpu.VMEM((_C, _H), jnp.float32),
        pltpu.VMEM((_ZR, _H), jnp.float32),
        pltpu.VMEM_SHARED((_N, _H), jnp.float32),
        pltpu.SemaphoreType.DMA,
    ],
)
def _scatter_kernel(g_hbm, src_hbm, dst_hbm, out_hbm,
                    src_v, dst_v, rows_v, zbuf_v, acc_sh, sem):
    c = lax.axis_index("c")
    s = lax.axis_index("s")
    wid = c * _NS + s
    base = s * _RA

    zero16 = jnp.zeros((16,), jnp.float32)

    def _fill_zero(i, carry):
        for j in range(_H // 16):
            zbuf_v[i, pl.ds(j * 16, 16)] = zero16
        return carry

    lax.fori_loop(0, _ZR, _fill_zero, 0)

    def _zero_copy(i, carry):
        pltpu.sync_copy(zbuf_v, acc_sh.at[pl.ds(base + i * _ZR, _ZR)])
        return carry

    lax.fori_loop(0, _RL // _ZR, _zero_copy, 0)

    @pl.when(s < _NS - 1)
    def _zero_tail():
        def _zc(i, carry):
            pltpu.sync_copy(zbuf_v,
                            acc_sh.at[pl.ds(base + _RL + i * _ZR, _ZR)])
            return carry
        lax.fori_loop(0, (_RA - _RL) // _ZR, _zc, 0)

    plsc.subcore_barrier()

    def _chunk(k, carry):
        off = wid * _EPW + k * _C
        pltpu.sync_copy(src_hbm.at[pl.ds(off, _C)], src_v)
        pltpu.sync_copy(dst_hbm.at[pl.ds(off, _C)], dst_v)
        pltpu.async_copy(g_hbm.at[src_v], rows_v, sem).wait()
        pltpu.sync_copy(rows_v, acc_sh.at[dst_v], add=True)
        return carry

    lax.fori_loop(0, _NCHUNK, _chunk, 0)

    plsc.subcore_barrier()
    pltpu.sync_copy(acc_sh.at[pl.ds(base, _RL)],
                    out_hbm.at[c, pl.ds(base, _RL)])

    @pl.when(s < _NS - 1)
    def _out_tail():
        pltpu.sync_copy(acc_sh.at[pl.ds(base + _RL, _RA - _RL)],
                        out_hbm.at[c, pl.ds(base + _RL, _RA - _RL)])


def _prologue_body(x_ref, we_ref, be_ref, w0_ref, deg_ref, g_ref, dis_ref):
    h = jnp.dot(x_ref[...], we_ref[...],
                preferred_element_type=jnp.float32) + be_ref[...]
    dsum = deg_ref[0] + deg_ref[1]
    dis = lax.rsqrt(dsum[:, 0:1] + 1.0)
    g_ref[...] = jnp.dot(h, w0_ref[...],
                         preferred_element_type=jnp.float32) * dis
    dis_ref[...] = dis


def _tc_prologue(x, w_enc, b_enc, w0, deg_pair):
    return pl.pallas_call(
        _prologue_body,
        grid=(_NB,),
        in_specs=[
            pl.BlockSpec((_BN, _H), lambda i: (i, 0)),
            pl.BlockSpec((_H, _H), lambda i: (0, 0)),
            pl.BlockSpec((1, _H), lambda i: (0, 0)),
            pl.BlockSpec((_H, _H), lambda i: (0, 0)),
            pl.BlockSpec((_NC, _BN, _H), lambda i: (0, i, 0)),
        ],
        out_specs=[
            pl.BlockSpec((_BN, _H), lambda i: (i, 0)),
            pl.BlockSpec((_BN, 1), lambda i: (i, 0)),
        ],
        out_shape=[
            jax.ShapeDtypeStruct((_N, _H), jnp.float32),
            jax.ShapeDtypeStruct((_N, 1), jnp.float32),
        ],
    )(x, w_enc, b_enc, w0, deg_pair)


def _mid_body(p_ref, g_ref, dis_ref, b_ref, w_ref, gn_ref):
    dis = dis_ref[...]
    ssum = (p_ref[0] + p_ref[1] + g_ref[...]) * dis + b_ref[...]
    h = jnp.maximum(ssum, 0.0)
    gn_ref[...] = jnp.dot(h, w_ref[...],
                          preferred_element_type=jnp.float32) * dis


def _tc_mid(p, g, dis, b_prev, w_next):
    return pl.pallas_call(
        _mid_body,
        grid=(_NB,),
        in_specs=[
            pl.BlockSpec((_NC, _BN, _H), lambda i: (0, i, 0)),
            pl.BlockSpec((_BN, _H), lambda i: (i, 0)),
            pl.BlockSpec((_BN, 1), lambda i: (i, 0)),
            pl.BlockSpec((1, _H), lambda i: (0, 0)),
            pl.BlockSpec((_H, _H), lambda i: (0, 0)),
        ],
        out_specs=pl.BlockSpec((_BN, _H), lambda i: (i, 0)),
        out_shape=jax.ShapeDtypeStruct((_N, _H), jnp.float32),
    )(p, g, dis, b_prev, w_next)


def _final_body(p_ref, g_ref, dis_ref, b_ref, batch_ref, wp_ref, bp_ref,
                lng_ref, lnb_ref, y_ref, sums_ref, cnt_ref):
    i = pl.program_id(0)

    @pl.when(i == 0)
    def _init():
        sums_ref[...] = jnp.zeros((_G, _H), jnp.float32)
        cnt_ref[...] = jnp.zeros((_G, 1), jnp.float32)

    dis = dis_ref[...]
    ssum = (p_ref[0] + p_ref[1] + g_ref[...]) * dis + b_ref[...]
    h = jnp.maximum(ssum, 0.0)

    b2d = batch_ref[0]
    gid = lax.broadcasted_iota(jnp.int32, (_G, _BN), 0)
    mask = (gid == b2d).astype(jnp.float32)
    sums_ref[...] += jnp.dot(mask, h, preferred_element_type=jnp.float32)
    cnt_ref[...] += jnp.sum(mask, axis=1, keepdims=True)

    @pl.when(i == _NB - 1)
    def _head():
        mol = sums_ref[...] / jnp.maximum(cnt_ref[...], 1.0)
        y = jnp.dot(mol, wp_ref[...],
                    preferred_element_type=jnp.float32) + bp_ref[...]
        mu = jnp.mean(y, axis=1, keepdims=True)
        var = jnp.mean((y - mu) * (y - mu), axis=1, keepdims=True)
        y = (y - mu) * lax.rsqrt(var + 1e-5)
        y_ref[...] = y * lng_ref[...] + lnb_ref[...]


def _tc_final(p, g, dis, b_prev, batch3, w_proj, b_proj, ln_g, ln_b):
    return pl.pallas_call(
        _final_body,
        grid=(_NB,),
        in_specs=[
            pl.BlockSpec((_NC, _BN, _H), lambda i: (0, i, 0)),
            pl.BlockSpec((_BN, _H), lambda i: (i, 0)),
            pl.BlockSpec((_BN, 1), lambda i: (i, 0)),
            pl.BlockSpec((1, _H), lambda i: (0, 0)),
            pl.BlockSpec((1, 1, _BN), lambda i: (i, 0, 0)),
            pl.BlockSpec((_H, _D), lambda i: (0, 0)),
            pl.BlockSpec((1, _D), lambda i: (0, 0)),
            pl.BlockSpec((1, _D), lambda i: (0, 0)),
            pl.BlockSpec((1, _D), lambda i: (0, 0)),
        ],
        out_specs=pl.BlockSpec((_G, _D), lambda i: (0, 0)),
        out_shape=jax.ShapeDtypeStruct((_G, _D), jnp.float32),
        scratch_shapes=[
            pltpu.VMEM((_G, _H), jnp.float32),
            pltpu.VMEM((_G, 1), jnp.float32),
        ],
    )(p, g, dis, b_prev, batch3, w_proj, b_proj, ln_g, ln_b)


def kernel(x, edge_index, batch, W_enc, b_enc, W_convs, b_convs,
           W_proj, b_proj, ln_g, ln_b):
    src = edge_index[0]
    dst = edge_index[1]

    deg_pair = _deg_kernel(dst)

    g1, dis = _tc_prologue(x, W_enc, b_enc.reshape(1, _H), W_convs[0],
                           deg_pair)
    p1 = _scatter_kernel(g1, src, dst)
    g2 = _tc_mid(p1, g1, dis, b_convs[0].reshape(1, _H), W_convs[1])
    p2 = _scatter_kernel(g2, src, dst)
    g3 = _tc_mid(p2, g2, dis, b_convs[1].reshape(1, _H), W_convs[2])
    p3 = _scatter_kernel(g3, src, dst)

    batch3 = batch.reshape(_NB, 1, _BN)
    y = _tc_final(p3, g3, dis, b_convs[2].reshape(1, _H), batch3,
                  W_proj, b_proj.reshape(1, _D),
                  ln_g.reshape(1, _D), ln_b.reshape(1, _D))
    return y


# speedup vs baseline: 20.8126x; 1.9195x over previous
import functools

import jax
import jax.numpy as jnp
from jax import lax
from jax.experimental import pallas as pl
from jax.experimental.pallas import tpu as pltpu
from jax.experimental.pallas import tpu_sc as plsc

_N = 10000
_H = 128
_E = 320000
_G = 64
_D = 64

_NC = 2
_NS = 16
_NW = _NC * _NS
_EPW = _E // _NW
_C = 80
_NCHUNK = _EPW // _C
_RA = 632
_RL = 520
_ZR = 8

_BN = 1000
_NB = _N // _BN

_mesh = plsc.VectorSubcoreMesh(core_axis_name="c", subcore_axis_name="s")


@functools.partial(
    pl.kernel,
    mesh=_mesh,
    out_type=jax.ShapeDtypeStruct((_NC, _N, _H), jnp.float32),
    scratch_types=[
        pltpu.VMEM((_C,), jnp.int32),
        pltpu.VMEM((_C,), jnp.int32),
        pltpu.VMEM((_C, _H), jnp.float32),
        pltpu.VMEM((_ZR, _H), jnp.float32),
        pltpu.VMEM_SHARED((_N, _H), jnp.float32),
        pltpu.SemaphoreType.DMA,
        pltpu.SemaphoreType.DMA,
    ],
)
def _deg_kernel(dst_hbm, out_hbm, dst_a, dst_b, ones_v, zbuf_v, acc_sh,
                sem_ia, sem_ib):
    c = lax.axis_index("c")
    s = lax.axis_index("s")
    wid = c * _NS + s
    base = s * _RA
    ebase = wid * _EPW

    one16 = jnp.ones((16,), jnp.float32)
    zero16 = jnp.zeros((16,), jnp.float32)

    def _fill_ones(i, carry):
        for j in range(_H // 16):
            ones_v[i, pl.ds(j * 16, 16)] = one16
        return carry

    lax.fori_loop(0, _C, _fill_ones, 0)

    def _fill_zero(i, carry):
        for j in range(_H // 16):
            zbuf_v[i, pl.ds(j * 16, 16)] = zero16
        return carry

    lax.fori_loop(0, _ZR, _fill_zero, 0)

    def _zero_copy(i, carry):
        pltpu.sync_copy(zbuf_v, acc_sh.at[pl.ds(base + i * _ZR, _ZR)])
        return carry

    lax.fori_loop(0, _RL // _ZR, _zero_copy, 0)

    @pl.when(s < _NS - 1)
    def _zero_tail():
        def _zc(i, carry):
            pltpu.sync_copy(zbuf_v,
                            acc_sh.at[pl.ds(base + _RL + i * _ZR, _ZR)])
            return carry
        lax.fori_loop(0, (_RA - _RL) // _ZR, _zc, 0)

    plsc.subcore_barrier()

    pltpu.async_copy(dst_hbm.at[pl.ds(ebase, _C)], dst_a, sem_ia).wait()
    pltpu.async_copy(dst_hbm.at[pl.ds(ebase + _C, _C)], dst_b, sem_ib)

    def _pair(j, carry):
        pltpu.sync_copy(ones_v, acc_sh.at[dst_a], add=True)
        pltpu.async_copy(dst_hbm.at[pl.ds(ebase + (2 * j + 2) * _C, _C)],
                         dst_a, sem_ia)
        pltpu.make_async_copy(dst_hbm.at[pl.ds(ebase + _C, _C)], dst_b,
                              sem_ib).wait()
        pltpu.sync_copy(ones_v, acc_sh.at[dst_b], add=True)

        @pl.when(2 * j + 3 < _NCHUNK)
        def _prefetch_b():
            pltpu.async_copy(dst_hbm.at[pl.ds(ebase + (2 * j + 3) * _C, _C)],
                             dst_b, sem_ib)

        pltpu.make_async_copy(dst_hbm.at[pl.ds(ebase, _C)], dst_a,
                              sem_ia).wait()
        return carry

    lax.fori_loop(0, (_NCHUNK - 1) // 2, _pair, 0)
    pltpu.sync_copy(ones_v, acc_sh.at[dst_a], add=True)

    plsc.subcore_barrier()
    pltpu.sync_copy(acc_sh.at[pl.ds(base, _RL)],
                    out_hbm.at[c, pl.ds(base, _RL)])

    @pl.when(s < _NS - 1)
    def _out_tail():
        pltpu.sync_copy(acc_sh.at[pl.ds(base + _RL, _RA - _RL)],
                        out_hbm.at[c, pl.ds(base + _RL, _RA - _RL)])


@functools.partial(
    pl.kernel,
    mesh=_mesh,
    out_type=jax.ShapeDtypeStruct((_NC, _N, _H), jnp.float32),
    scratch_types=[
        pltpu.VMEM((_C,), jnp.int32),
        pltpu.VMEM((_C,), jnp.int32),
        pltpu.VMEM((_C,), jnp.int32),
        pltpu.VMEM((_C,), jnp.int32),
        pltpu.VMEM((_C, _H), jnp.float32),
        pltpu.VMEM((_C, _H), jnp.float32),
        pltpu.VMEM((_ZR, _H), jnp.float32),
        pltpu.VMEM_SHARED((_N, _H), jnp.float32),
        pltpu.SemaphoreType.DMA,
        pltpu.SemaphoreType.DMA,
        pltpu.SemaphoreType.DMA,
        pltpu.SemaphoreType.DMA,
    ],
)
def _scatter_kernel(g_hbm, src_hbm, dst_hbm, out_hbm,
                    src_a, dst_a, src_b, dst_b, rows_a, rows_b,
                    zbuf_v, acc_sh, sem_a, sem_b, sem_ia, sem_ib):
    c = lax.axis_index("c")
    s = lax.axis_index("s")
    wid = c * _NS + s
    base = s * _RA
    ebase = wid * _EPW

    zero16 = jnp.zeros((16,), jnp.float32)

    def _fill_zero(i, carry):
        for j in range(_H // 16):
            zbuf_v[i, pl.ds(j * 16, 16)] = zero16
        return carry

    lax.fori_loop(0, _ZR, _fill_zero, 0)

    def _zero_copy(i, carry):
        pltpu.sync_copy(zbuf_v, acc_sh.at[pl.ds(base + i * _ZR, _ZR)])
        return carry

    lax.fori_loop(0, _RL // _ZR, _zero_copy, 0)

    @pl.when(s < _NS - 1)
    def _zero_tail():
        def _zc(i, carry):
            pltpu.sync_copy(zbuf_v,
                            acc_sh.at[pl.ds(base + _RL + i * _ZR, _ZR)])
            return carry
        lax.fori_loop(0, (_RA - _RL) // _ZR, _zc, 0)

    plsc.subcore_barrier()

    def _load_idx(k, sv, dv, sem):
        off = ebase + k * _C
        h1 = pltpu.async_copy(src_hbm.at[pl.ds(off, _C)], sv, sem)
        h2 = pltpu.async_copy(dst_hbm.at[pl.ds(off, _C)], dv, sem)
        return h1, h2

    h1, h2 = _load_idx(0, src_a, dst_a, sem_ia)
    h1.wait(); h2.wait()
    ga = pltpu.async_copy(g_hbm.at[src_a], rows_a, sem_a)
    h1, h2 = _load_idx(1, src_b, dst_b, sem_ib)
    h1.wait(); h2.wait()

    def _pair(j, carry):
        gb = pltpu.async_copy(g_hbm.at[src_b], rows_b, sem_b)
        pltpu.make_async_copy(g_hbm.at[src_a], rows_a, sem_a).wait()
        pltpu.sync_copy(rows_a, acc_sh.at[dst_a], add=True)
        i1, i2 = _load_idx(2 * j + 2, src_a, dst_a, sem_ia)
        i1.wait(); i2.wait()
        pltpu.async_copy(g_hbm.at[src_a], rows_a, sem_a)
        pltpu.make_async_copy(g_hbm.at[src_b], rows_b, sem_b).wait()
        pltpu.sync_copy(rows_b, acc_sh.at[dst_b], add=True)

        @pl.when(2 * j + 3 < _NCHUNK)
        def _prefetch_b():
            i3, i4 = _load_idx(2 * j + 3, src_b, dst_b, sem_ib)
            i3.wait(); i4.wait()
        return carry

    lax.fori_loop(0, (_NCHUNK - 1) // 2, _pair, 0)

    pltpu.make_async_copy(g_hbm.at[src_a], rows_a, sem_a).wait()
    pltpu.sync_copy(rows_a, acc_sh.at[dst_a], add=True)

    plsc.subcore_barrier()
    pltpu.sync_copy(acc_sh.at[pl.ds(base, _RL)],
                    out_hbm.at[c, pl.ds(base, _RL)])

    @pl.when(s < _NS - 1)
    def _out_tail():
        pltpu.sync_copy(acc_sh.at[pl.ds(base + _RL, _RA - _RL)],
                        out_hbm.at[c, pl.ds(base + _RL, _RA - _RL)])


def _prologue_body(x_ref, we_ref, be_ref, w0_ref, deg_ref, g_ref, dis_ref):
    h = jnp.dot(x_ref[...], we_ref[...],
                preferred_element_type=jnp.float32) + be_ref[...]
    dsum = deg_ref[0] + deg_ref[1]
    dis = lax.rsqrt(dsum[:, 0:1] + 1.0)
    g_ref[...] = jnp.dot(h, w0_ref[...],
                         preferred_element_type=jnp.float32) * dis
    dis_ref[...] = dis


def _tc_prologue(x, w_enc, b_enc, w0, deg_pair):
    return pl.pallas_call(
        _prologue_body,
        grid=(_NB,),
        in_specs=[
            pl.BlockSpec((_BN, _H), lambda i: (i, 0)),
            pl.BlockSpec((_H, _H), lambda i: (0, 0)),
            pl.BlockSpec((1, _H), lambda i: (0, 0)),
            pl.BlockSpec((_H, _H), lambda i: (0, 0)),
            pl.BlockSpec((_NC, _BN, _H), lambda i: (0, i, 0)),
        ],
        out_specs=[
            pl.BlockSpec((_BN, _H), lambda i: (i, 0)),
            pl.BlockSpec((_BN, 1), lambda i: (i, 0)),
        ],
        out_shape=[
            jax.ShapeDtypeStruct((_N, _H), jnp.float32),
            jax.ShapeDtypeStruct((_N, 1), jnp.float32),
        ],
    )(x, w_enc, b_enc, w0, deg_pair)


def _mid_body(p_ref, g_ref, dis_ref, b_ref, w_ref, gn_ref):
    dis = dis_ref[...]
    ssum = (p_ref[0] + p_ref[1] + g_ref[...]) * dis + b_ref[...]
    h = jnp.maximum(ssum, 0.0)
    gn_ref[...] = jnp.dot(h, w_ref[...],
                          preferred_element_type=jnp.float32) * dis


def _tc_mid(p, g, dis, b_prev, w_next):
    return pl.pallas_call(
        _mid_body,
        grid=(_NB,),
        in_specs=[
            pl.BlockSpec((_NC, _BN, _H), lambda i: (0, i, 0)),
            pl.BlockSpec((_BN, _H), lambda i: (i, 0)),
            pl.BlockSpec((_BN, 1), lambda i: (i, 0)),
            pl.BlockSpec((1, _H), lambda i: (0, 0)),
            pl.BlockSpec((_H, _H), lambda i: (0, 0)),
        ],
        out_specs=pl.BlockSpec((_BN, _H), lambda i: (i, 0)),
        out_shape=jax.ShapeDtypeStruct((_N, _H), jnp.float32),
    )(p, g, dis, b_prev, w_next)


def _final_body(p_ref, g_ref, dis_ref, b_ref, batch_ref, wp_ref, bp_ref,
                lng_ref, lnb_ref, y_ref, sums_ref, cnt_ref):
    i = pl.program_id(0)

    @pl.when(i == 0)
    def _init():
        sums_ref[...] = jnp.zeros((_G, _H), jnp.float32)
        cnt_ref[...] = jnp.zeros((_G, 1), jnp.float32)

    dis = dis_ref[...]
    ssum = (p_ref[0] + p_ref[1] + g_ref[...]) * dis + b_ref[...]
    h = jnp.maximum(ssum, 0.0)

    b2d = batch_ref[0]
    gid = lax.broadcasted_iota(jnp.int32, (_G, _BN), 0)
    mask = (gid == b2d).astype(jnp.float32)
    sums_ref[...] += jnp.dot(mask, h, preferred_element_type=jnp.float32)
    cnt_ref[...] += jnp.sum(mask, axis=1, keepdims=True)

    @pl.when(i == _NB - 1)
    def _head():
        mol = sums_ref[...] / jnp.maximum(cnt_ref[...], 1.0)
        y = jnp.dot(mol, wp_ref[...],
                    preferred_element_type=jnp.float32) + bp_ref[...]
        mu = jnp.mean(y, axis=1, keepdims=True)
        var = jnp.mean((y - mu) * (y - mu), axis=1, keepdims=True)
        y = (y - mu) * lax.rsqrt(var + 1e-5)
        y_ref[...] = y * lng_ref[...] + lnb_ref[...]


def _tc_final(p, g, dis, b_prev, batch3, w_proj, b_proj, ln_g, ln_b):
    return pl.pallas_call(
        _final_body,
        grid=(_NB,),
        in_specs=[
            pl.BlockSpec((_NC, _BN, _H), lambda i: (0, i, 0)),
            pl.BlockSpec((_BN, _H), lambda i: (i, 0)),
            pl.BlockSpec((_BN, 1), lambda i: (i, 0)),
            pl.BlockSpec((1, _H), lambda i: (0, 0)),
            pl.BlockSpec((1, 1, _BN), lambda i: (i, 0, 0)),
            pl.BlockSpec((_H, _D), lambda i: (0, 0)),
            pl.BlockSpec((1, _D), lambda i: (0, 0)),
            pl.BlockSpec((1, _D), lambda i: (0, 0)),
            pl.BlockSpec((1, _D), lambda i: (0, 0)),
        ],
        out_specs=pl.BlockSpec((_G, _D), lambda i: (0, 0)),
        out_shape=jax.ShapeDtypeStruct((_G, _D), jnp.float32),
        scratch_shapes=[
            pltpu.VMEM((_G, _H), jnp.float32),
            pltpu.VMEM((_G, 1), jnp.float32),
        ],
    )(p, g, dis, b_prev, batch3, w_proj, b_proj, ln_g, ln_b)


def kernel(x, edge_index, batch, W_enc, b_enc, W_convs, b_convs,
           W_proj, b_proj, ln_g, ln_b):
    src = edge_index[0]
    dst = edge_index[1]

    deg_pair = _deg_kernel(dst)

    g1, dis = _tc_prologue(x, W_enc, b_enc.reshape(1, _H), W_convs[0],
                           deg_pair)
    p1 = _scatter_kernel(g1, src, dst)
    g2 = _tc_mid(p1, g1, dis, b_convs[0].reshape(1, _H), W_convs[1])
    p2 = _scatter_kernel(g2, src, dst)
    g3 = _tc_mid(p2, g2, dis, b_convs[1].reshape(1, _H), W_convs[2])
    p3 = _scatter_kernel(g3, src, dst)

    batch3 = batch.reshape(_NB, 1, _BN)
    y = _tc_final(p3, g3, dis, b_convs[2].reshape(1, _H), batch3,
                  W_proj, b_proj.reshape(1, _D),
                  ln_g.reshape(1, _D), ln_b.reshape(1, _D))
    return y


# speedup vs baseline: 24.3195x; 1.1685x over previous
import functools

import jax
import jax.numpy as jnp
from jax import lax
from jax.experimental import pallas as pl
from jax.experimental.pallas import tpu as pltpu
from jax.experimental.pallas import tpu_sc as plsc

_N = 10000
_H = 128
_E = 320000
_G = 64
_D = 64

_NC = 2
_NS = 16
_NW = _NC * _NS
_EPW = _E // _NW
_C = 80
_NCHUNK = _EPW // _C
_RA = 632
_RL = 520
_ZR = 8

_BN = 1000
_NB = _N // _BN

_mesh = plsc.VectorSubcoreMesh(core_axis_name="c", subcore_axis_name="s")


@functools.partial(
    pl.kernel,
    mesh=_mesh,
    out_type=jax.ShapeDtypeStruct((_NC, _N, _H), jnp.float32),
    scratch_types=[
        pltpu.VMEM((_NCHUNK, _C), jnp.int32),
        pltpu.VMEM((_C, _H), jnp.float32),
        pltpu.VMEM((_ZR, _H), jnp.float32),
        pltpu.VMEM_SHARED((_N, _H), jnp.float32),
        pltpu.SemaphoreType.DMA,
    ],
)
def _deg_kernel(dst_hbm, out_hbm, dst_t, ones_v, zbuf_v, acc_sh, sem_i):
    c = lax.axis_index("c")
    s = lax.axis_index("s")
    wid = c * _NS + s
    base = s * _RA

    hi = pltpu.async_copy(dst_hbm.at[wid], dst_t, sem_i)

    one16 = jnp.ones((16,), jnp.float32)
    zero16 = jnp.zeros((16,), jnp.float32)

    def _fill_ones(i, carry):
        for j in range(_H // 16):
            ones_v[i, pl.ds(j * 16, 16)] = one16
        return carry

    lax.fori_loop(0, _C, _fill_ones, 0)

    def _fill_zero(i, carry):
        for j in range(_H // 16):
            zbuf_v[i, pl.ds(j * 16, 16)] = zero16
        return carry

    lax.fori_loop(0, _ZR, _fill_zero, 0)

    def _zero_copy(i, carry):
        pltpu.sync_copy(zbuf_v, acc_sh.at[pl.ds(base + i * _ZR, _ZR)])
        return carry

    lax.fori_loop(0, _RL // _ZR, _zero_copy, 0)

    @pl.when(s < _NS - 1)
    def _zero_tail():
        def _zc(i, carry):
            pltpu.sync_copy(zbuf_v,
                            acc_sh.at[pl.ds(base + _RL + i * _ZR, _ZR)])
            return carry
        lax.fori_loop(0, (_RA - _RL) // _ZR, _zc, 0)

    plsc.subcore_barrier()
    hi.wait()

    def _chunk(k, carry):
        pltpu.sync_copy(ones_v, acc_sh.at[dst_t.at[k]], add=True)
        return carry

    lax.fori_loop(0, _NCHUNK, _chunk, 0)

    plsc.subcore_barrier()
    pltpu.sync_copy(acc_sh.at[pl.ds(base, _RL)],
                    out_hbm.at[c, pl.ds(base, _RL)])

    @pl.when(s < _NS - 1)
    def _out_tail():
        pltpu.sync_copy(acc_sh.at[pl.ds(base + _RL, _RA - _RL)],
                        out_hbm.at[c, pl.ds(base + _RL, _RA - _RL)])


@functools.partial(
    pl.kernel,
    mesh=_mesh,
    out_type=jax.ShapeDtypeStruct((_NC, _N, _H), jnp.float32),
    scratch_types=[
        pltpu.VMEM((_NCHUNK, _C), jnp.int32),
        pltpu.VMEM((_C,), jnp.int32),
        pltpu.VMEM((_C,), jnp.int32),
        pltpu.VMEM((_C, _H), jnp.float32),
        pltpu.VMEM((_C, _H), jnp.float32),
        pltpu.VMEM((_ZR, _H), jnp.float32),
        pltpu.VMEM_SHARED((_N, _H), jnp.float32),
        pltpu.SemaphoreType.DMA,
        pltpu.SemaphoreType.DMA,
        pltpu.SemaphoreType.DMA,
        pltpu.SemaphoreType.DMA,
        pltpu.SemaphoreType.DMA,
    ],
)
def _scatter_kernel(g_hbm, src_hbm, dst_hbm_flat, out_hbm,
                    src_t, dst_a, dst_b, rows_0, rows_1,
                    zbuf_v, acc_sh, sem_0, sem_1, sem_da, sem_db, sem_i):
    c = lax.axis_index("c")
    s = lax.axis_index("s")
    wid = c * _NS + s
    base = s * _RA
    ebase = wid * _EPW
    rows = (rows_0, rows_1)
    sems = (sem_0, sem_1)
    dsts = (dst_a, dst_b)
    dsems = (sem_da, sem_db)

    hi1 = pltpu.async_copy(src_hbm.at[wid], src_t, sem_i)

    zero16 = jnp.zeros((16,), jnp.float32)

    def _fill_zero(i, carry):
        for j in range(_H // 16):
            zbuf_v[i, pl.ds(j * 16, 16)] = zero16
        return carry

    lax.fori_loop(0, _ZR, _fill_zero, 0)

    def _zero_copy(i, carry):
        pltpu.sync_copy(zbuf_v, acc_sh.at[pl.ds(base + i * _ZR, _ZR)])
        return carry

    lax.fori_loop(0, _RL // _ZR, _zero_copy, 0)

    @pl.when(s < _NS - 1)
    def _zero_tail():
        def _zc(i, carry):
            pltpu.sync_copy(zbuf_v,
                            acc_sh.at[pl.ds(base + _RL + i * _ZR, _ZR)])
            return carry
        lax.fori_loop(0, (_RA - _RL) // _ZR, _zc, 0)

    plsc.subcore_barrier()
    hi1.wait()

    def _dst_load(k, b):
        return pltpu.async_copy(
            dst_hbm_flat.at[pl.ds(ebase + k * _C, _C)], dsts[b], dsems[b])

    for b in range(2):
        pltpu.async_copy(g_hbm.at[src_t.at[b]], rows[b], sems[b])
        _dst_load(b, b)

    def _group(j, carry):
        for b in range(2):
            k = 2 * j + b
            pltpu.make_async_copy(g_hbm.at[src_t.at[k]], rows[b],
                                  sems[b]).wait()
            pltpu.make_async_copy(
                dst_hbm_flat.at[pl.ds(ebase + k * _C, _C)], dsts[b],
                dsems[b]).wait()
            pltpu.sync_copy(rows[b], acc_sh.at[dsts[b]], add=True)

            @pl.when(k + 2 < _NCHUNK)
            def _refill():
                pltpu.async_copy(g_hbm.at[src_t.at[k + 2]], rows[b],
                                 sems[b])
                _dst_load(k + 2, b)
        return carry

    lax.fori_loop(0, _NCHUNK // 2, _group, 0)

    k = _NCHUNK - 1
    b = k % 2
    pltpu.make_async_copy(g_hbm.at[src_t.at[k]], rows[b], sems[b]).wait()
    pltpu.make_async_copy(dst_hbm_flat.at[pl.ds(ebase + k * _C, _C)],
                          dsts[b], dsems[b]).wait()
    pltpu.sync_copy(rows[b], acc_sh.at[dsts[b]], add=True)

    plsc.subcore_barrier()
    pltpu.sync_copy(acc_sh.at[pl.ds(base, _RL)],
                    out_hbm.at[c, pl.ds(base, _RL)])

    @pl.when(s < _NS - 1)
    def _out_tail():
        pltpu.sync_copy(acc_sh.at[pl.ds(base + _RL, _RA - _RL)],
                        out_hbm.at[c, pl.ds(base + _RL, _RA - _RL)])


def _prologue_body(x_ref, we_ref, be_ref, w0_ref, deg_ref, g_ref, dis_ref):
    h = jnp.dot(x_ref[...], we_ref[...],
                preferred_element_type=jnp.float32) + be_ref[...]
    dsum = deg_ref[0] + deg_ref[1]
    dis = lax.rsqrt(dsum[:, 0:1] + 1.0)
    g_ref[...] = jnp.dot(h, w0_ref[...],
                         preferred_element_type=jnp.float32) * dis
    dis_ref[...] = dis


def _tc_prologue(x, w_enc, b_enc, w0, deg_pair):
    return pl.pallas_call(
        _prologue_body,
        grid=(_NB,),
        in_specs=[
            pl.BlockSpec((_BN, _H), lambda i: (i, 0)),
            pl.BlockSpec((_H, _H), lambda i: (0, 0)),
            pl.BlockSpec((1, _H), lambda i: (0, 0)),
            pl.BlockSpec((_H, _H), lambda i: (0, 0)),
            pl.BlockSpec((_NC, _BN, _H), lambda i: (0, i, 0)),
        ],
        out_specs=[
            pl.BlockSpec((_BN, _H), lambda i: (i, 0)),
            pl.BlockSpec((_BN, 1), lambda i: (i, 0)),
        ],
        out_shape=[
            jax.ShapeDtypeStruct((_N, _H), jnp.float32),
            jax.ShapeDtypeStruct((_N, 1), jnp.float32),
        ],
    )(x, w_enc, b_enc, w0, deg_pair)


def _mid_body(p_ref, g_ref, dis_ref, b_ref, w_ref, gn_ref):
    dis = dis_ref[...]
    ssum = (p_ref[0] + p_ref[1] + g_ref[...]) * dis + b_ref[...]
    h = jnp.maximum(ssum, 0.0)
    gn_ref[...] = jnp.dot(h, w_ref[...],
                          preferred_element_type=jnp.float32) * dis


def _tc_mid(p, g, dis, b_prev, w_next):
    return pl.pallas_call(
        _mid_body,
        grid=(_NB,),
        in_specs=[
            pl.BlockSpec((_NC, _BN, _H), lambda i: (0, i, 0)),
            pl.BlockSpec((_BN, _H), lambda i: (i, 0)),
            pl.BlockSpec((_BN, 1), lambda i: (i, 0)),
            pl.BlockSpec((1, _H), lambda i: (0, 0)),
            pl.BlockSpec((_H, _H), lambda i: (0, 0)),
        ],
        out_specs=pl.BlockSpec((_BN, _H), lambda i: (i, 0)),
        out_shape=jax.ShapeDtypeStruct((_N, _H), jnp.float32),
    )(p, g, dis, b_prev, w_next)


def _final_body(p_ref, g_ref, dis_ref, b_ref, batch_ref, wp_ref, bp_ref,
                lng_ref, lnb_ref, y_ref, sums_ref, cnt_ref):
    i = pl.program_id(0)

    @pl.when(i == 0)
    def _init():
        sums_ref[...] = jnp.zeros((_G, _H), jnp.float32)
        cnt_ref[...] = jnp.zeros((_G, 1), jnp.float32)

    dis = dis_ref[...]
    ssum = (p_ref[0] + p_ref[1] + g_ref[...]) * dis + b_ref[...]
    h = jnp.maximum(ssum, 0.0)

    b2d = batch_ref[0]
    gid = lax.broadcasted_iota(jnp.int32, (_G, _BN), 0)
    mask = (gid == b2d).astype(jnp.float32)
    sums_ref[...] += jnp.dot(mask, h, preferred_element_type=jnp.float32)
    cnt_ref[...] += jnp.sum(mask, axis=1, keepdims=True)

    @pl.when(i == _NB - 1)
    def _head():
        mol = sums_ref[...] / jnp.maximum(cnt_ref[...], 1.0)
        y = jnp.dot(mol, wp_ref[...],
                    preferred_element_type=jnp.float32) + bp_ref[...]
        mu = jnp.mean(y, axis=1, keepdims=True)
        var = jnp.mean((y - mu) * (y - mu), axis=1, keepdims=True)
        y = (y - mu) * lax.rsqrt(var + 1e-5)
        y_ref[...] = y * lng_ref[...] + lnb_ref[...]


def _tc_final(p, g, dis, b_prev, batch3, w_proj, b_proj, ln_g, ln_b):
    return pl.pallas_call(
        _final_body,
        grid=(_NB,),
        in_specs=[
            pl.BlockSpec((_NC, _BN, _H), lambda i: (0, i, 0)),
            pl.BlockSpec((_BN, _H), lambda i: (i, 0)),
            pl.BlockSpec((_BN, 1), lambda i: (i, 0)),
            pl.BlockSpec((1, _H), lambda i: (0, 0)),
            pl.BlockSpec((1, 1, _BN), lambda i: (i, 0, 0)),
            pl.BlockSpec((_H, _D), lambda i: (0, 0)),
            pl.BlockSpec((1, _D), lambda i: (0, 0)),
            pl.BlockSpec((1, _D), lambda i: (0, 0)),
            pl.BlockSpec((1, _D), lambda i: (0, 0)),
        ],
        out_specs=pl.BlockSpec((_G, _D), lambda i: (0, 0)),
        out_shape=jax.ShapeDtypeStruct((_G, _D), jnp.float32),
        scratch_shapes=[
            pltpu.VMEM((_G, _H), jnp.float32),
            pltpu.VMEM((_G, 1), jnp.float32),
        ],
    )(p, g, dis, b_prev, batch3, w_proj, b_proj, ln_g, ln_b)


def kernel(x, edge_index, batch, W_enc, b_enc, W_convs, b_convs,
           W_proj, b_proj, ln_g, ln_b):
    src = edge_index[0].reshape(_NW, _NCHUNK, _C)
    dst_flat = edge_index[1]
    dst = dst_flat.reshape(_NW, _NCHUNK, _C)

    deg_pair = _deg_kernel(dst)

    g1, dis = _tc_prologue(x, W_enc, b_enc.reshape(1, _H), W_convs[0],
                           deg_pair)
    p1 = _scatter_kernel(g1, src, dst_flat)
    g2 = _tc_mid(p1, g1, dis, b_convs[0].reshape(1, _H), W_convs[1])
    p2 = _scatter_kernel(g2, src, dst_flat)
    g3 = _tc_mid(p2, g2, dis, b_convs[1].reshape(1, _H), W_convs[2])
    p3 = _scatter_kernel(g3, src, dst_flat)

    batch3 = batch.reshape(_NB, 1, _BN)
    y = _tc_final(p3, g3, dis, b_convs[2].reshape(1, _H), batch3,
                  W_proj, b_proj.reshape(1, _D),
                  ln_g.reshape(1, _D), ln_b.reshape(1, _D))
    return y


# speedup vs baseline: 26.4413x; 1.0872x over previous
import functools

import jax
import jax.numpy as jnp
from jax import lax
from jax.experimental import pallas as pl
from jax.experimental.pallas import tpu as pltpu
from jax.experimental.pallas import tpu_sc as plsc

_N = 10000
_H = 128
_E = 320000
_G = 64
_D = 64

_NC = 2
_NS = 16
_NW = _NC * _NS
_EPW = _E // _NW
_C = 80
_NCHUNK = _EPW // _C
_RA = 632
_RL = 520
_ZR = 8

_DW = 16
_BN = 1000
_NB = _N // _BN

_mesh = plsc.VectorSubcoreMesh(core_axis_name="c", subcore_axis_name="s")


@functools.partial(
    pl.kernel,
    mesh=_mesh,
    out_type=jax.ShapeDtypeStruct((_NC, _N, _DW), jnp.float32),
    scratch_types=[
        pltpu.VMEM((_NCHUNK, _C), jnp.int32),
        pltpu.VMEM((_C, _DW), jnp.float32),
        pltpu.VMEM((_ZR, _DW), jnp.float32),
        pltpu.VMEM_SHARED((_N, _DW), jnp.float32),
        pltpu.SemaphoreType.DMA,
    ],
)
def _deg_kernel(dst_hbm, out_hbm, dst_t, ones_v, zbuf_v, acc_sh, sem_i):
    c = lax.axis_index("c")
    s = lax.axis_index("s")
    wid = c * _NS + s
    base = s * _RA

    hi = pltpu.async_copy(dst_hbm.at[wid], dst_t, sem_i)

    one16 = jnp.ones((16,), jnp.float32)
    zero16 = jnp.zeros((16,), jnp.float32)

    def _fill_ones(i, carry):
        for j in range(_DW // 16):
            ones_v[i, pl.ds(j * 16, 16)] = one16
        return carry

    lax.fori_loop(0, _C, _fill_ones, 0)

    def _fill_zero(i, carry):
        for j in range(_DW // 16):
            zbuf_v[i, pl.ds(j * 16, 16)] = zero16
        return carry

    lax.fori_loop(0, _ZR, _fill_zero, 0)

    def _zero_copy(i, carry):
        pltpu.sync_copy(zbuf_v, acc_sh.at[pl.ds(base + i * _ZR, _ZR)])
        return carry

    lax.fori_loop(0, _RL // _ZR, _zero_copy, 0)

    @pl.when(s < _NS - 1)
    def _zero_tail():
        def _zc(i, carry):
            pltpu.sync_copy(zbuf_v,
                            acc_sh.at[pl.ds(base + _RL + i * _ZR, _ZR)])
            return carry
        lax.fori_loop(0, (_RA - _RL) // _ZR, _zc, 0)

    plsc.subcore_barrier()
    hi.wait()

    def _chunk(k, carry):
        pltpu.sync_copy(ones_v, acc_sh.at[dst_t.at[k]], add=True)
        return carry

    lax.fori_loop(0, _NCHUNK, _chunk, 0)

    plsc.subcore_barrier()
    pltpu.sync_copy(acc_sh.at[pl.ds(base, _RL)],
                    out_hbm.at[c, pl.ds(base, _RL)])

    @pl.when(s < _NS - 1)
    def _out_tail():
        pltpu.sync_copy(acc_sh.at[pl.ds(base + _RL, _RA - _RL)],
                        out_hbm.at[c, pl.ds(base + _RL, _RA - _RL)])


@functools.partial(
    pl.kernel,
    mesh=_mesh,
    out_type=jax.ShapeDtypeStruct((_NC, _N, _H), jnp.float32),
    scratch_types=[
        pltpu.VMEM((_NCHUNK, _C), jnp.int32),
        pltpu.VMEM((_C,), jnp.int32),
        pltpu.VMEM((_C,), jnp.int32),
        pltpu.VMEM((_C, _H), jnp.float32),
        pltpu.VMEM((_C, _H), jnp.float32),
        pltpu.VMEM((_ZR, _H), jnp.float32),
        pltpu.VMEM_SHARED((_N, _H), jnp.float32),
        pltpu.SemaphoreType.DMA,
        pltpu.SemaphoreType.DMA,
        pltpu.SemaphoreType.DMA,
        pltpu.SemaphoreType.DMA,
        pltpu.SemaphoreType.DMA,
    ],
)
def _scatter_kernel(g_hbm, src_hbm, dst_hbm_flat, out_hbm,
                    src_t, dst_a, dst_b, rows_0, rows_1,
                    zbuf_v, acc_sh, sem_0, sem_1, sem_da, sem_db, sem_i):
    c = lax.axis_index("c")
    s = lax.axis_index("s")
    wid = c * _NS + s
    base = s * _RA
    ebase = wid * _EPW
    rows = (rows_0, rows_1)
    sems = (sem_0, sem_1)
    dsts = (dst_a, dst_b)
    dsems = (sem_da, sem_db)

    hi1 = pltpu.async_copy(src_hbm.at[wid], src_t, sem_i)

    zero16 = jnp.zeros((16,), jnp.float32)

    def _fill_zero(i, carry):
        for j in range(_H // 16):
            zbuf_v[i, pl.ds(j * 16, 16)] = zero16
        return carry

    lax.fori_loop(0, _ZR, _fill_zero, 0)

    def _zero_copy(i, carry):
        pltpu.sync_copy(zbuf_v, acc_sh.at[pl.ds(base + i * _ZR, _ZR)])
        return carry

    lax.fori_loop(0, _RL // _ZR, _zero_copy, 0)

    @pl.when(s < _NS - 1)
    def _zero_tail():
        def _zc(i, carry):
            pltpu.sync_copy(zbuf_v,
                            acc_sh.at[pl.ds(base + _RL + i * _ZR, _ZR)])
            return carry
        lax.fori_loop(0, (_RA - _RL) // _ZR, _zc, 0)

    plsc.subcore_barrier()
    hi1.wait()

    def _dst_load(k, b):
        return pltpu.async_copy(
            dst_hbm_flat.at[pl.ds(ebase + k * _C, _C)], dsts[b], dsems[b])

    for b in range(2):
        pltpu.async_copy(g_hbm.at[src_t.at[b]], rows[b], sems[b])
        _dst_load(b, b)

    def _group(j, carry):
        for b in range(2):
            k = 2 * j + b
            pltpu.make_async_copy(g_hbm.at[src_t.at[k]], rows[b],
                                  sems[b]).wait()
            pltpu.make_async_copy(
                dst_hbm_flat.at[pl.ds(ebase + k * _C, _C)], dsts[b],
                dsems[b]).wait()
            pltpu.sync_copy(rows[b], acc_sh.at[dsts[b]], add=True)

            @pl.when(k + 2 < _NCHUNK)
            def _refill():
                pltpu.async_copy(g_hbm.at[src_t.at[k + 2]], rows[b],
                                 sems[b])
                _dst_load(k + 2, b)
        return carry

    lax.fori_loop(0, _NCHUNK // 2, _group, 0)

    k = _NCHUNK - 1
    b = k % 2
    pltpu.make_async_copy(g_hbm.at[src_t.at[k]], rows[b], sems[b]).wait()
    pltpu.make_async_copy(dst_hbm_flat.at[pl.ds(ebase + k * _C, _C)],
                          dsts[b], dsems[b]).wait()
    pltpu.sync_copy(rows[b], acc_sh.at[dsts[b]], add=True)

    plsc.subcore_barrier()
    pltpu.sync_copy(acc_sh.at[pl.ds(base, _RL)],
                    out_hbm.at[c, pl.ds(base, _RL)])

    @pl.when(s < _NS - 1)
    def _out_tail():
        pltpu.sync_copy(acc_sh.at[pl.ds(base + _RL, _RA - _RL)],
                        out_hbm.at[c, pl.ds(base + _RL, _RA - _RL)])


def _prologue_body(x_ref, we_ref, be_ref, w0_ref, deg_ref, g_ref, dis_ref):
    h = jnp.dot(x_ref[...], we_ref[...],
                preferred_element_type=jnp.float32) + be_ref[...]
    dsum = deg_ref[0] + deg_ref[1]
    dis = lax.rsqrt(dsum[:, 0:1] + 1.0)
    g_ref[...] = jnp.dot(h, w0_ref[...],
                         preferred_element_type=jnp.float32) * dis
    dis_ref[...] = dis


def _tc_prologue(x, w_enc, b_enc, w0, deg_pair):
    return pl.pallas_call(
        _prologue_body,
        grid=(_NB,),
        in_specs=[
            pl.BlockSpec((_BN, _H), lambda i: (i, 0)),
            pl.BlockSpec((_H, _H), lambda i: (0, 0)),
            pl.BlockSpec((1, _H), lambda i: (0, 0)),
            pl.BlockSpec((_H, _H), lambda i: (0, 0)),
            pl.BlockSpec((_NC, _BN, _DW), lambda i: (0, i, 0)),
        ],
        out_specs=[
            pl.BlockSpec((_BN, _H), lambda i: (i, 0)),
            pl.BlockSpec((_BN, 1), lambda i: (i, 0)),
        ],
        out_shape=[
            jax.ShapeDtypeStruct((_N, _H), jnp.float32),
            jax.ShapeDtypeStruct((_N, 1), jnp.float32),
        ],
    )(x, w_enc, b_enc, w0, deg_pair)


def _mid_body(p_ref, g_ref, dis_ref, b_ref, w_ref, gn_ref):
    dis = dis_ref[...]
    ssum = (p_ref[0] + p_ref[1] + g_ref[...]) * dis + b_ref[...]
    h = jnp.maximum(ssum, 0.0)
    gn_ref[...] = jnp.dot(h, w_ref[...],
                          preferred_element_type=jnp.float32) * dis


def _tc_mid(p, g, dis, b_prev, w_next):
    return pl.pallas_call(
        _mid_body,
        grid=(_NB,),
        in_specs=[
            pl.BlockSpec((_NC, _BN, _H), lambda i: (0, i, 0)),
            pl.BlockSpec((_BN, _H), lambda i: (i, 0)),
            pl.BlockSpec((_BN, 1), lambda i: (i, 0)),
            pl.BlockSpec((1, _H), lambda i: (0, 0)),
            pl.BlockSpec((_H, _H), lambda i: (0, 0)),
        ],
        out_specs=pl.BlockSpec((_BN, _H), lambda i: (i, 0)),
        out_shape=jax.ShapeDtypeStruct((_N, _H), jnp.float32),
    )(p, g, dis, b_prev, w_next)


def _final_body(p_ref, g_ref, dis_ref, b_ref, batch_ref, wp_ref, bp_ref,
                lng_ref, lnb_ref, y_ref, sums_ref, cnt_ref):
    i = pl.program_id(0)

    @pl.when(i == 0)
    def _init():
        sums_ref[...] = jnp.zeros((_G, _H), jnp.float32)
        cnt_ref[...] = jnp.zeros((_G, 1), jnp.float32)

    dis = dis_ref[...]
    ssum = (p_ref[0] + p_ref[1] + g_ref[...]) * dis + b_ref[...]
    h = jnp.maximum(ssum, 0.0)

    b2d = batch_ref[0]
    gid = lax.broadcasted_iota(jnp.int32, (_G, _BN), 0)
    mask = (gid == b2d).astype(jnp.float32)
    sums_ref[...] += jnp.dot(mask, h, preferred_element_type=jnp.float32)
    cnt_ref[...] += jnp.sum(mask, axis=1, keepdims=True)

    @pl.when(i == _NB - 1)
    def _head():
        mol = sums_ref[...] / jnp.maximum(cnt_ref[...], 1.0)
        y = jnp.dot(mol, wp_ref[...],
                    preferred_element_type=jnp.float32) + bp_ref[...]
        mu = jnp.mean(y, axis=1, keepdims=True)
        var = jnp.mean((y - mu) * (y - mu), axis=1, keepdims=True)
        y = (y - mu) * lax.rsqrt(var + 1e-5)
        y_ref[...] = y * lng_ref[...] + lnb_ref[...]


def _tc_final(p, g, dis, b_prev, batch3, w_proj, b_proj, ln_g, ln_b):
    return pl.pallas_call(
        _final_body,
        grid=(_NB,),
        in_specs=[
            pl.BlockSpec((_NC, _BN, _H), lambda i: (0, i, 0)),
            pl.BlockSpec((_BN, _H), lambda i: (i, 0)),
            pl.BlockSpec((_BN, 1), lambda i: (i, 0)),
            pl.BlockSpec((1, _H), lambda i: (0, 0)),
            pl.BlockSpec((1, 1, _BN), lambda i: (i, 0, 0)),
            pl.BlockSpec((_H, _D), lambda i: (0, 0)),
            pl.BlockSpec((1, _D), lambda i: (0, 0)),
            pl.BlockSpec((1, _D), lambda i: (0, 0)),
            pl.BlockSpec((1, _D), lambda i: (0, 0)),
        ],
        out_specs=pl.BlockSpec((_G, _D), lambda i: (0, 0)),
        out_shape=jax.ShapeDtypeStruct((_G, _D), jnp.float32),
        scratch_shapes=[
            pltpu.VMEM((_G, _H), jnp.float32),
            pltpu.VMEM((_G, 1), jnp.float32),
        ],
    )(p, g, dis, b_prev, batch3, w_proj, b_proj, ln_g, ln_b)


def kernel(x, edge_index, batch, W_enc, b_enc, W_convs, b_convs,
           W_proj, b_proj, ln_g, ln_b):
    src = edge_index[0].reshape(_NW, _NCHUNK, _C)
    dst_flat = edge_index[1]
    dst = dst_flat.reshape(_NW, _NCHUNK, _C)

    deg_pair = _deg_kernel(dst)

    g1, dis = _tc_prologue(x, W_enc, b_enc.reshape(1, _H), W_convs[0],
                           deg_pair)
    p1 = _scatter_kernel(g1, src, dst_flat)
    g2 = _tc_mid(p1, g1, dis, b_convs[0].reshape(1, _H), W_convs[1])
    p2 = _scatter_kernel(g2, src, dst_flat)
    g3 = _tc_mid(p2, g2, dis, b_convs[1].reshape(1, _H), W_convs[2])
    p3 = _scatter_kernel(g3, src, dst_flat)

    batch3 = batch.reshape(_NB, 1, _BN)
    y = _tc_final(p3, g3, dis, b_convs[2].reshape(1, _H), batch3,
                  W_proj, b_proj.reshape(1, _D),
                  ln_g.reshape(1, _D), ln_b.reshape(1, _D))
    return y


# speedup vs baseline: 30.9046x; 1.1688x over previous
import functools

import jax
import jax.numpy as jnp
from jax import lax
from jax.experimental import pallas as pl
from jax.experimental.pallas import tpu as pltpu
from jax.experimental.pallas import tpu_sc as plsc

_N = 10000
_H = 128
_E = 320000
_G = 64
_D = 64

_NC = 2
_NS = 16
_NW = _NC * _NS
_EPW = _E // _NW
_C = 80
_NCHUNK = _EPW // _C
_RA = 632
_RL = 520
_ZR = 8

_NR = 3
_DW = 16
_BN = 1000
_NB = _N // _BN

_mesh = plsc.VectorSubcoreMesh(core_axis_name="c", subcore_axis_name="s")


@functools.partial(
    pl.kernel,
    mesh=_mesh,
    out_type=jax.ShapeDtypeStruct((_NC, _N, _DW), jnp.float32),
    scratch_types=[
        pltpu.VMEM((_NCHUNK, _C), jnp.int32),
        pltpu.VMEM((_C, _DW), jnp.float32),
        pltpu.VMEM((_ZR, _DW), jnp.float32),
        pltpu.VMEM_SHARED((_N, _DW), jnp.float32),
        pltpu.SemaphoreType.DMA,
    ],
)
def _deg_kernel(dst_hbm, out_hbm, dst_t, ones_v, zbuf_v, acc_sh, sem_i):
    c = lax.axis_index("c")
    s = lax.axis_index("s")
    wid = c * _NS + s
    base = s * _RA

    hi = pltpu.async_copy(dst_hbm.at[wid], dst_t, sem_i)

    one16 = jnp.ones((16,), jnp.float32)
    zero16 = jnp.zeros((16,), jnp.float32)

    def _fill_ones(i, carry):
        for j in range(_DW // 16):
            ones_v[i, pl.ds(j * 16, 16)] = one16
        return carry

    lax.fori_loop(0, _C, _fill_ones, 0)

    def _fill_zero(i, carry):
        for j in range(_DW // 16):
            zbuf_v[i, pl.ds(j * 16, 16)] = zero16
        return carry

    lax.fori_loop(0, _ZR, _fill_zero, 0)

    def _zero_copy(i, carry):
        pltpu.sync_copy(zbuf_v, acc_sh.at[pl.ds(base + i * _ZR, _ZR)])
        return carry

    lax.fori_loop(0, _RL // _ZR, _zero_copy, 0)

    @pl.when(s < _NS - 1)
    def _zero_tail():
        def _zc(i, carry):
            pltpu.sync_copy(zbuf_v,
                            acc_sh.at[pl.ds(base + _RL + i * _ZR, _ZR)])
            return carry
        lax.fori_loop(0, (_RA - _RL) // _ZR, _zc, 0)

    plsc.subcore_barrier()
    hi.wait()

    def _chunk(k, carry):
        pltpu.sync_copy(ones_v, acc_sh.at[dst_t.at[k]], add=True)
        return carry

    lax.fori_loop(0, _NCHUNK, _chunk, 0)

    plsc.subcore_barrier()
    pltpu.sync_copy(acc_sh.at[pl.ds(base, _RL)],
                    out_hbm.at[c, pl.ds(base, _RL)])

    @pl.when(s < _NS - 1)
    def _out_tail():
        pltpu.sync_copy(acc_sh.at[pl.ds(base + _RL, _RA - _RL)],
                        out_hbm.at[c, pl.ds(base + _RL, _RA - _RL)])


@functools.partial(
    pl.kernel,
    mesh=_mesh,
    out_type=jax.ShapeDtypeStruct((_NC, _N, _H), jnp.float32),
    scratch_types=[
        pltpu.VMEM((_NCHUNK, _C), jnp.int32),
    ] + [pltpu.VMEM((_C,), jnp.int32) for _ in range(_NR)]
      + [pltpu.VMEM((_C, _H), jnp.float32) for _ in range(_NR)]
      + [
        pltpu.VMEM((_ZR, _H), jnp.float32),
        pltpu.VMEM_SHARED((_N, _H), jnp.float32),
    ] + [pltpu.SemaphoreType.DMA for _ in range(2 * _NR + 1)],
)
def _scatter_kernel(g_hbm, src_hbm, dst_hbm_flat, out_hbm, src_t, *refs):
    dsts = refs[:_NR]
    rows = refs[_NR:2 * _NR]
    zbuf_v = refs[2 * _NR]
    acc_sh = refs[2 * _NR + 1]
    sems = refs[2 * _NR + 2:3 * _NR + 2]
    dsems = refs[3 * _NR + 2:4 * _NR + 2]
    sem_i = refs[4 * _NR + 2]
    c = lax.axis_index("c")
    s = lax.axis_index("s")
    wid = c * _NS + s
    base = s * _RA
    ebase = wid * _EPW

    hi1 = pltpu.async_copy(src_hbm.at[wid], src_t, sem_i)

    zero16 = jnp.zeros((16,), jnp.float32)

    def _fill_zero(i, carry):
        for j in range(_H // 16):
            zbuf_v[i, pl.ds(j * 16, 16)] = zero16
        return carry

    lax.fori_loop(0, _ZR, _fill_zero, 0)

    def _zero_copy(i, carry):
        pltpu.sync_copy(zbuf_v, acc_sh.at[pl.ds(base + i * _ZR, _ZR)])
        return carry

    lax.fori_loop(0, _RL // _ZR, _zero_copy, 0)

    @pl.when(s < _NS - 1)
    def _zero_tail():
        def _zc(i, carry):
            pltpu.sync_copy(zbuf_v,
                            acc_sh.at[pl.ds(base + _RL + i * _ZR, _ZR)])
            return carry
        lax.fori_loop(0, (_RA - _RL) // _ZR, _zc, 0)

    plsc.subcore_barrier()
    hi1.wait()

    def _dst_load(k, b):
        return pltpu.async_copy(
            dst_hbm_flat.at[pl.ds(ebase + k * _C, _C)], dsts[b], dsems[b])

    for b in range(_NR):
        pltpu.async_copy(g_hbm.at[src_t.at[b]], rows[b], sems[b])
        _dst_load(b, b)

    def _group(j, carry):
        for b in range(_NR):
            k = _NR * j + b
            pltpu.make_async_copy(g_hbm.at[src_t.at[k]], rows[b],
                                  sems[b]).wait()
            pltpu.make_async_copy(
                dst_hbm_flat.at[pl.ds(ebase + k * _C, _C)], dsts[b],
                dsems[b]).wait()
            pltpu.sync_copy(rows[b], acc_sh.at[dsts[b]], add=True)

            @pl.when(k + _NR < _NCHUNK)
            def _refill():
                pltpu.async_copy(g_hbm.at[src_t.at[k + _NR]], rows[b],
                                 sems[b])
                _dst_load(k + _NR, b)
        return carry

    lax.fori_loop(0, _NCHUNK // _NR, _group, 0)

    for k in range((_NCHUNK // _NR) * _NR, _NCHUNK):
        b = k % _NR
        pltpu.make_async_copy(g_hbm.at[src_t.at[k]], rows[b],
                              sems[b]).wait()
        pltpu.make_async_copy(dst_hbm_flat.at[pl.ds(ebase + k * _C, _C)],
                              dsts[b], dsems[b]).wait()
        pltpu.sync_copy(rows[b], acc_sh.at[dsts[b]], add=True)

    plsc.subcore_barrier()
    pltpu.sync_copy(acc_sh.at[pl.ds(base, _RL)],
                    out_hbm.at[c, pl.ds(base, _RL)])

    @pl.when(s < _NS - 1)
    def _out_tail():
        pltpu.sync_copy(acc_sh.at[pl.ds(base + _RL, _RA - _RL)],
                        out_hbm.at[c, pl.ds(base + _RL, _RA - _RL)])


def _prologue_body(x_ref, we_ref, be_ref, w0_ref, deg_ref, g_ref, dis_ref):
    h = jnp.dot(x_ref[...], we_ref[...],
                preferred_element_type=jnp.float32) + be_ref[...]
    dsum = deg_ref[0] + deg_ref[1]
    dis = lax.rsqrt(dsum[:, 0:1] + 1.0)
    g_ref[...] = jnp.dot(h, w0_ref[...],
                         preferred_element_type=jnp.float32) * dis
    dis_ref[...] = dis


def _tc_prologue(x, w_enc, b_enc, w0, deg_pair):
    return pl.pallas_call(
        _prologue_body,
        grid=(_NB,),
        in_specs=[
            pl.BlockSpec((_BN, _H), lambda i: (i, 0)),
            pl.BlockSpec((_H, _H), lambda i: (0, 0)),
            pl.BlockSpec((1, _H), lambda i: (0, 0)),
            pl.BlockSpec((_H, _H), lambda i: (0, 0)),
            pl.BlockSpec((_NC, _BN, _DW), lambda i: (0, i, 0)),
        ],
        out_specs=[
            pl.BlockSpec((_BN, _H), lambda i: (i, 0)),
            pl.BlockSpec((_BN, 1), lambda i: (i, 0)),
        ],
        out_shape=[
            jax.ShapeDtypeStruct((_N, _H), jnp.float32),
            jax.ShapeDtypeStruct((_N, 1), jnp.float32),
        ],
    )(x, w_enc, b_enc, w0, deg_pair)


def _mid_body(p_ref, g_ref, dis_ref, b_ref, w_ref, gn_ref):
    dis = dis_ref[...]
    ssum = (p_ref[0] + p_ref[1] + g_ref[...]) * dis + b_ref[...]
    h = jnp.maximum(ssum, 0.0)
    gn_ref[...] = jnp.dot(h, w_ref[...],
                          preferred_element_type=jnp.float32) * dis


def _tc_mid(p, g, dis, b_prev, w_next):
    return pl.pallas_call(
        _mid_body,
        grid=(_NB,),
        in_specs=[
            pl.BlockSpec((_NC, _BN, _H), lambda i: (0, i, 0)),
            pl.BlockSpec((_BN, _H), lambda i: (i, 0)),
            pl.BlockSpec((_BN, 1), lambda i: (i, 0)),
            pl.BlockSpec((1, _H), lambda i: (0, 0)),
            pl.BlockSpec((_H, _H), lambda i: (0, 0)),
        ],
        out_specs=pl.BlockSpec((_BN, _H), lambda i: (i, 0)),
        out_shape=jax.ShapeDtypeStruct((_N, _H), jnp.float32),
    )(p, g, dis, b_prev, w_next)


def _final_body(p_ref, g_ref, dis_ref, b_ref, batch_ref, wp_ref, bp_ref,
                lng_ref, lnb_ref, y_ref, sums_ref, cnt_ref):
    i = pl.program_id(0)

    @pl.when(i == 0)
    def _init():
        sums_ref[...] = jnp.zeros((_G, _H), jnp.float32)
        cnt_ref[...] = jnp.zeros((_G, 1), jnp.float32)

    dis = dis_ref[...]
    ssum = (p_ref[0] + p_ref[1] + g_ref[...]) * dis + b_ref[...]
    h = jnp.maximum(ssum, 0.0)

    b2d = batch_ref[0]
    gid = lax.broadcasted_iota(jnp.int32, (_G, _BN), 0)
    mask = (gid == b2d).astype(jnp.float32)
    sums_ref[...] += jnp.dot(mask, h, preferred_element_type=jnp.float32)
    cnt_ref[...] += jnp.sum(mask, axis=1, keepdims=True)

    @pl.when(i == _NB - 1)
    def _head():
        mol = sums_ref[...] / jnp.maximum(cnt_ref[...], 1.0)
        y = jnp.dot(mol, wp_ref[...],
                    preferred_element_type=jnp.float32) + bp_ref[...]
        mu = jnp.mean(y, axis=1, keepdims=True)
        var = jnp.mean((y - mu) * (y - mu), axis=1, keepdims=True)
        y = (y - mu) * lax.rsqrt(var + 1e-5)
        y_ref[...] = y * lng_ref[...] + lnb_ref[...]


def _tc_final(p, g, dis, b_prev, batch3, w_proj, b_proj, ln_g, ln_b):
    return pl.pallas_call(
        _final_body,
        grid=(_NB,),
        in_specs=[
            pl.BlockSpec((_NC, _BN, _H), lambda i: (0, i, 0)),
            pl.BlockSpec((_BN, _H), lambda i: (i, 0)),
            pl.BlockSpec((_BN, 1), lambda i: (i, 0)),
            pl.BlockSpec((1, _H), lambda i: (0, 0)),
            pl.BlockSpec((1, 1, _BN), lambda i: (i, 0, 0)),
            pl.BlockSpec((_H, _D), lambda i: (0, 0)),
            pl.BlockSpec((1, _D), lambda i: (0, 0)),
            pl.BlockSpec((1, _D), lambda i: (0, 0)),
            pl.BlockSpec((1, _D), lambda i: (0, 0)),
        ],
        out_specs=pl.BlockSpec((_G, _D), lambda i: (0, 0)),
        out_shape=jax.ShapeDtypeStruct((_G, _D), jnp.float32),
        scratch_shapes=[
            pltpu.VMEM((_G, _H), jnp.float32),
            pltpu.VMEM((_G, 1), jnp.float32),
        ],
    )(p, g, dis, b_prev, batch3, w_proj, b_proj, ln_g, ln_b)


def kernel(x, edge_index, batch, W_enc, b_enc, W_convs, b_convs,
           W_proj, b_proj, ln_g, ln_b):
    src = edge_index[0].reshape(_NW, _NCHUNK, _C)
    dst_flat = edge_index[1]
    dst = dst_flat.reshape(_NW, _NCHUNK, _C)

    deg_pair = _deg_kernel(dst)

    g1, dis = _tc_prologue(x, W_enc, b_enc.reshape(1, _H), W_convs[0],
                           deg_pair)
    p1 = _scatter_kernel(g1, src, dst_flat)
    g2 = _tc_mid(p1, g1, dis, b_convs[0].reshape(1, _H), W_convs[1])
    p2 = _scatter_kernel(g2, src, dst_flat)
    g3 = _tc_mid(p2, g2, dis, b_convs[1].reshape(1, _H), W_convs[2])
    p3 = _scatter_kernel(g3, src, dst_flat)

    batch3 = batch.reshape(_NB, 1, _BN)
    y = _tc_final(p3, g3, dis, b_convs[2].reshape(1, _H), batch3,
                  W_proj, b_proj.reshape(1, _D),
                  ln_g.reshape(1, _D), ln_b.reshape(1, _D))
    return y
